# Initial kernel scaffold; baseline (speedup 1.0000x reference)
import jax
import jax.numpy as jnp
from jax.experimental import pallas as pl


def kernel(z, edge_index, W, b):
    raise NotImplementedError("write your pallas kernel here")


# speedup vs baseline: 24.5881x; 24.5881x over previous
import functools

import jax
import jax.numpy as jnp
from jax import lax
from jax.experimental import pallas as pl
from jax.experimental.pallas import tpu as pltpu
from jax.experimental.pallas import tpu_sc as plsc

_NC = 2
_NS = 16
_NW = _NC * _NS
_L = 16


def _matvec_body(z_ref, w_ref, b_ref, y_ref):
    y_ref[...] = (
        jnp.sum(z_ref[...] * w_ref[...], axis=1, keepdims=True)
        + b_ref[...] * 0.5
    )


def _edge_body(y_hbm, src_hbm, dst_hbm, out_hbm, y_v, src_v, dst_v, out_v):
    n = y_v.shape[0]
    ew = src_v.shape[0]
    wid = lax.axis_index("s") * _NC + lax.axis_index("c")
    base = wid * ew
    pltpu.sync_copy(y_hbm, y_v)
    pltpu.sync_copy(src_hbm.at[pl.ds(base, ew)], src_v)
    pltpu.sync_copy(dst_hbm.at[pl.ds(base, ew)], dst_v)

    def step(i, carry):
        off = i * _L
        s_idx = src_v[pl.ds(off, _L)]
        d_idx = dst_v[pl.ds(off, _L)]
        sv = plsc.load_gather(y_v, [s_idx])
        dv = plsc.load_gather(y_v, [d_idx])
        x = sv + dv
        out_v[pl.ds(off, _L)] = 1.0 / (1.0 + jnp.exp(-x))
        return carry

    lax.fori_loop(0, ew // _L, step, 0)
    pltpu.sync_copy(out_v, out_hbm.at[pl.ds(base, ew)])


def kernel(z, edge_index, W, b):
    n, d = z.shape
    e = edge_index.shape[1]
    ew = e // _NW

    y = pl.pallas_call(
        _matvec_body,
        out_shape=jax.ShapeDtypeStruct((n, 1), jnp.float32),
    )(z, W, b.reshape(1, 1))
    y_flat = y.reshape(n)

    ei = edge_index.astype(jnp.int32)

    edge_kernel = pl.kernel(
        _edge_body,
        out_type=jax.ShapeDtypeStruct((e,), jnp.float32),
        mesh=plsc.VectorSubcoreMesh(
            core_axis_name="c", subcore_axis_name="s"
        ),
        compiler_params=pltpu.CompilerParams(needs_layout_passes=False),
        scratch_types=[
            pltpu.VMEM((n,), jnp.float32),
            pltpu.VMEM((ew,), jnp.int32),
            pltpu.VMEM((ew,), jnp.int32),
            pltpu.VMEM((ew,), jnp.float32),
        ],
    )
    out = edge_kernel(y_flat, ei[0], ei[1])
    return out.reshape(e, 1)


# speedup vs baseline: 31.7124x; 1.2897x over previous
import functools

import jax
import jax.numpy as jnp
from jax import lax
from jax.experimental import pallas as pl
from jax.experimental.pallas import tpu as pltpu
from jax.experimental.pallas import tpu_sc as plsc

_NC = 2
_NS = 16
_NW = _NC * _NS
_L = 16


def _matvec_body(z_ref, w_ref, b_ref, y_ref):
    y_ref[...] = jnp.sum(z_ref[...] * w_ref[...], axis=1) + b_ref[0, 0] * 0.5


def _edge_body(y_hbm, src_hbm, dst_hbm, out_hbm, y_v, src_v, dst_v, out_v):
    ew = src_v.shape[0]
    wid = lax.axis_index("s") * _NC + lax.axis_index("c")
    base = wid * ew
    pltpu.sync_copy(y_hbm, y_v)
    pltpu.sync_copy(src_hbm.at[pl.ds(base, ew)], src_v)
    pltpu.sync_copy(dst_hbm.at[pl.ds(base, ew)], dst_v)

    @plsc.parallel_loop(0, ew, _L, unroll=8)
    def step(off):
        sv = plsc.load_gather(y_v, [src_v[pl.ds(off, _L)]])
        dv = plsc.load_gather(y_v, [dst_v[pl.ds(off, _L)]])
        x = sv + dv
        out_v[pl.ds(off, _L)] = 1.0 / (1.0 + jnp.exp(-x))

    pltpu.sync_copy(out_v, out_hbm.at[pl.ds(base, ew)])


def kernel(z, edge_index, W, b):
    n, d = z.shape
    e = edge_index.shape[1]
    ew = e // _NW

    y = pl.pallas_call(
        _matvec_body,
        out_shape=jax.ShapeDtypeStruct((n,), jnp.float32),
    )(z, W, b.reshape(1, 1))

    ei = edge_index.astype(jnp.int32)

    edge_kernel = pl.kernel(
        _edge_body,
        out_type=jax.ShapeDtypeStruct((e,), jnp.float32),
        mesh=plsc.VectorSubcoreMesh(
            core_axis_name="c", subcore_axis_name="s"
        ),
        compiler_params=pltpu.CompilerParams(needs_layout_passes=False),
        scratch_types=[
            pltpu.VMEM((n,), jnp.float32),
            pltpu.VMEM((ew,), jnp.int32),
            pltpu.VMEM((ew,), jnp.int32),
            pltpu.VMEM((ew,), jnp.float32),
        ],
    )
    out = edge_kernel(y, ei[0], ei[1])
    return out.reshape(e, 1)


# speedup vs baseline: 39.0383x; 1.2310x over previous
import functools

import jax
import jax.numpy as jnp
from jax import lax
from jax.experimental import pallas as pl
from jax.experimental.pallas import tpu as pltpu
from jax.experimental.pallas import tpu_sc as plsc

_NC = 2
_NS = 16
_NW = _NC * _NS
_L = 16


def _matvec_body(z_ref, w_ref, b_ref, y_ref):
    y_ref[...] = jnp.sum(z_ref[...] * w_ref[...], axis=1) + b_ref[0, 0] * 0.5


def _edge_body(y_hbm, ei_hbm, out_hbm, y_v, src_v, dst_v, out_v):
    ew = src_v.shape[0]
    e = ei_hbm.shape[0] // 2
    wid = lax.axis_index("s") * _NC + lax.axis_index("c")
    base = wid * ew
    pltpu.sync_copy(y_hbm, y_v)
    pltpu.sync_copy(ei_hbm.at[pl.ds(base, ew)], src_v)
    pltpu.sync_copy(ei_hbm.at[pl.ds(e + base, ew)], dst_v)

    @plsc.parallel_loop(0, ew, _L, unroll=8)
    def step(off):
        sv = plsc.load_gather(y_v, [src_v[pl.ds(off, _L)]])
        dv = plsc.load_gather(y_v, [dst_v[pl.ds(off, _L)]])
        x = sv + dv
        out_v[pl.ds(off, _L)] = 1.0 / (1.0 + jnp.exp(-x))

    pltpu.sync_copy(out_v, out_hbm.at[pl.ds(base, ew)])


def kernel(z, edge_index, W, b):
    n, d = z.shape
    e = edge_index.shape[1]
    ew = e // _NW

    y = pl.pallas_call(
        _matvec_body,
        out_shape=jax.ShapeDtypeStruct((n,), jnp.float32),
    )(z, W, b.reshape(1, 1))

    ei = edge_index.astype(jnp.int32).reshape(2 * e)

    edge_kernel = pl.kernel(
        _edge_body,
        out_type=jax.ShapeDtypeStruct((e,), jnp.float32),
        mesh=plsc.VectorSubcoreMesh(
            core_axis_name="c", subcore_axis_name="s"
        ),
        compiler_params=pltpu.CompilerParams(needs_layout_passes=False),
        scratch_types=[
            pltpu.VMEM((n,), jnp.float32),
            pltpu.VMEM((ew,), jnp.int32),
            pltpu.VMEM((ew,), jnp.int32),
            pltpu.VMEM((ew,), jnp.float32),
        ],
    )
    out = edge_kernel(y, ei)
    return out.reshape(e, 1)


# speedup vs baseline: 44.9981x; 1.1527x over previous
import functools

import jax
import jax.numpy as jnp
from jax import lax
from jax.experimental import pallas as pl
from jax.experimental.pallas import tpu as pltpu
from jax.experimental.pallas import tpu_sc as plsc

_NC = 2
_NS = 16
_NW = _NC * _NS
_L = 16
_TILE = 128


def _matvec_body(z_ref, w_ref, b_ref, y_ref):
    y_ref[...] = jnp.sum(z_ref[...] * w_ref[...], axis=1) + b_ref[0, 0] * 0.5


def _edge_body(y_hbm, ei_hbm, out_hbm, y_v, idx_v, out_v):
    cw = idx_v.shape[1]
    e = ei_hbm.shape[1]
    ntiles = e // _TILE
    wid = lax.axis_index("s") * _NC + lax.axis_index("c")
    base = (wid * ntiles) // _NW * _TILE
    pltpu.sync_copy(y_hbm, y_v)
    pltpu.sync_copy(ei_hbm.at[:, pl.ds(base, cw)], idx_v)

    @plsc.parallel_loop(0, cw, _L, unroll=8)
    def step(off):
        sv = plsc.load_gather(y_v, [idx_v[0, pl.ds(off, _L)]])
        dv = plsc.load_gather(y_v, [idx_v[1, pl.ds(off, _L)]])
        x = sv + dv
        out_v[pl.ds(off, _L)] = 1.0 / (1.0 + jnp.exp(-x))

    pltpu.sync_copy(out_v, out_hbm.at[pl.ds(base, cw)])


def kernel(z, edge_index, W, b):
    n, d = z.shape
    e = edge_index.shape[1]
    ntiles = e // _TILE
    cw = (ntiles + _NW - 1) // _NW * _TILE

    y = pl.pallas_call(
        _matvec_body,
        out_shape=jax.ShapeDtypeStruct((n,), jnp.float32),
    )(z, W, b.reshape(1, 1))

    ei = edge_index.astype(jnp.int32)

    edge_kernel = pl.kernel(
        _edge_body,
        out_type=jax.ShapeDtypeStruct((e,), jnp.float32),
        mesh=plsc.VectorSubcoreMesh(
            core_axis_name="c", subcore_axis_name="s"
        ),
        compiler_params=pltpu.CompilerParams(needs_layout_passes=False),
        scratch_types=[
            pltpu.VMEM((n,), jnp.float32),
            pltpu.VMEM((2, cw), jnp.int32),
            pltpu.VMEM((cw,), jnp.float32),
        ],
    )
    out = edge_kernel(y, ei)
    return out.reshape(e, 1)
